# baseline (speedup 1.0000x reference)
import functools

import jax
import jax.numpy as jnp
from jax import lax
from jax.experimental import pallas as pl
from jax.experimental.pallas import tpu as pltpu
from jax.experimental.pallas import tpu_sc as plsc

B, N, D = 16384, 200, 128


def _sc_gather(emb_flat, idx):
    info = plsc.get_sparse_core_info()
    NC, NS, L = info.num_cores, info.num_subcores, info.num_lanes
    NW = NC * NS
    b_per_w = B // NW
    CH = 128
    n_ch = b_per_w // CH
    mesh = plsc.VectorSubcoreMesh(core_axis_name="c", subcore_axis_name="s")

    @functools.partial(
        pl.kernel,
        mesh=mesh,
        out_type=jax.ShapeDtypeStruct((B, D), jnp.float32),
        scratch_types=[
            pltpu.VMEM((b_per_w,), jnp.int32),
            pltpu.VMEM((n_ch, CH), jnp.int32),
            pltpu.VMEM((b_per_w, D), jnp.float32),
            pltpu.SemaphoreType.DMA,
        ],
    )
    def k(emb_hbm, idx_hbm, out_hbm, idx_raw, idx_v, rows_v, sem):
        wid = lax.axis_index("s") * NC + lax.axis_index("c")
        base = wid * b_per_w
        pltpu.sync_copy(idx_hbm.at[pl.ds(base, b_per_w)], idx_raw)
        lane_off = lax.iota(jnp.int32, L) * N
        for j in range(b_per_w // L):
            row0 = (base + j * L) * N
            v = idx_raw[pl.ds(j * L, L)] + (lane_off + row0)
            idx_v[(j * L) // CH, pl.ds((j * L) % CH, L)] = v
        copies = []
        for c in range(n_ch):
            cp = pltpu.make_async_copy(
                emb_hbm.at[idx_v.at[c]], rows_v.at[pl.ds(c * CH, CH)], sem
            )
            cp.start()
            copies.append(cp)
        for cp in copies:
            cp.wait()
        pltpu.sync_copy(rows_v, out_hbm.at[pl.ds(base, b_per_w)])

    return k(emb_flat, idx)


def _tc_project(g, vc, uc, ct, ii, w0, wf, bias):
    BLK = 1024
    grid = (B // BLK,)

    def body(g_ref, vc_ref, uc_ref, ct_ref, ii_ref, w0_ref, wf_ref, b_ref, o_ref):
        acc = jnp.dot(g_ref[...], w0_ref[...], preferred_element_type=jnp.float32)
        rc = vc_ref[...] - uc_ref[...]
        acc += rc * wf_ref[0:1, :]
        acc += ct_ref[...] * wf_ref[1:2, :]
        acc += ii_ref[...] * wf_ref[2:3, :]
        o_ref[...] = acc + b_ref[...]

    row = lambda i: (i, 0)
    fixed = lambda i: (0, 0)
    return pl.pallas_call(
        body,
        grid=grid,
        in_specs=[
            pl.BlockSpec((BLK, D), row),
            pl.BlockSpec((BLK, 1), row),
            pl.BlockSpec((BLK, 1), row),
            pl.BlockSpec((BLK, 1), row),
            pl.BlockSpec((BLK, 1), row),
            pl.BlockSpec((D, D), fixed),
            pl.BlockSpec((3, D), fixed),
            pl.BlockSpec((1, D), fixed),
        ],
        out_specs=pl.BlockSpec((BLK, D), row),
        out_shape=jax.ShapeDtypeStruct((B, D), jnp.float32),
    )(g, vc, uc, ct, ii, w0, wf, bias)


def kernel(embeddings, current_node, vehicle_capacity, used_capacity, current_time, i, W, b):
    emb_flat = embeddings.reshape(B * N, D)
    idx = current_node.astype(jnp.int32)
    g = jax.lax.slice(emb_flat, (0, 0), (B, D))
    w0 = W[:D]
    wf = W[D:]
    bias = b.reshape(1, D)
    return _tc_project(g, vehicle_capacity, used_capacity, current_time, i, w0, wf, bias)


# speedup vs baseline: 1.2104x; 1.2104x over previous
import functools

import jax
import jax.numpy as jnp
from jax import lax
from jax.experimental import pallas as pl
from jax.experimental.pallas import tpu as pltpu
from jax.experimental.pallas import tpu_sc as plsc

B, N, D = 16384, 200, 128


def _sc_gather(emb_flat, idx):
    info = plsc.get_sparse_core_info()
    NC, NS, L = info.num_cores, info.num_subcores, info.num_lanes
    NW = NC * NS
    b_per_w = B // NW
    CH = 128
    n_ch = b_per_w // CH
    mesh = plsc.VectorSubcoreMesh(core_axis_name="c", subcore_axis_name="s")

    @functools.partial(
        pl.kernel,
        mesh=mesh,
        out_type=jax.ShapeDtypeStruct((B, D), jnp.float32),
        scratch_types=[
            pltpu.VMEM((b_per_w,), jnp.int32),
            pltpu.VMEM((n_ch, CH), jnp.int32),
            pltpu.VMEM((b_per_w, D), jnp.float32),
            pltpu.SemaphoreType.DMA,
        ],
    )
    def k(emb_hbm, idx_hbm, out_hbm, idx_raw, idx_v, rows_v, sem):
        wid = lax.axis_index("s") * NC + lax.axis_index("c")
        base = wid * b_per_w
        pltpu.sync_copy(idx_hbm.at[pl.ds(base, b_per_w)], idx_raw)
        lane_off = lax.iota(jnp.int32, L) * N
        for j in range(b_per_w // L):
            row0 = (base + j * L) * N
            v = idx_raw[pl.ds(j * L, L)] + (lane_off + row0)
            idx_v[(j * L) // CH, pl.ds((j * L) % CH, L)] = v
        copies = []
        for c in range(n_ch):
            cp = pltpu.make_async_copy(
                emb_hbm.at[idx_v.at[c]], rows_v.at[pl.ds(c * CH, CH)], sem
            )
            cp.start()
            copies.append(cp)
        for cp in copies:
            cp.wait()
        pltpu.sync_copy(rows_v, out_hbm.at[pl.ds(base, b_per_w)])

    return k(emb_flat, idx)


def _tc_project(g, ft, w0, wfx, bias):
    BLK = 1024
    grid = (B // BLK,)

    def body(g_ref, ft_ref, w0_ref, wfx_ref, b_ref, o_ref):
        acc = jnp.dot(g_ref[...], w0_ref[...], preferred_element_type=jnp.float32)
        acc += lax.dot_general(
            ft_ref[...], wfx_ref[...], (((0,), (0,)), ((), ())),
            preferred_element_type=jnp.float32,
        )
        o_ref[...] = acc + b_ref[...]

    row = lambda i: (i, 0)
    fcol = lambda i: (0, i)
    fixed = lambda i: (0, 0)
    return pl.pallas_call(
        body,
        grid=grid,
        in_specs=[
            pl.BlockSpec((BLK, D), row),
            pl.BlockSpec((4, BLK), fcol),
            pl.BlockSpec((D, D), fixed),
            pl.BlockSpec((4, D), fixed),
            pl.BlockSpec((1, D), fixed),
        ],
        out_specs=pl.BlockSpec((BLK, D), row),
        out_shape=jax.ShapeDtypeStruct((B, D), jnp.float32),
    )(g, ft, w0, wfx, bias)


def kernel(embeddings, current_node, vehicle_capacity, used_capacity, current_time, i, W, b):
    emb_flat = embeddings.reshape(B * N, D)
    idx = current_node.astype(jnp.int32)
    g = _sc_gather(emb_flat, idx)
    w0 = W[:D]
    ft = jnp.concatenate(
        [vehicle_capacity.T, used_capacity.T, current_time.T, i.T], axis=0
    )
    wfx = jnp.concatenate([W[D:D + 1], -W[D:D + 1], W[D + 1:D + 2], W[D + 2:D + 3]], axis=0)
    bias = b.reshape(1, D)
    return _tc_project(g, ft, w0, wfx, bias)


# speedup vs baseline: 1.3415x; 1.1083x over previous
import functools

import jax
import jax.numpy as jnp
from jax import lax
from jax.experimental import pallas as pl
from jax.experimental.pallas import tpu as pltpu
from jax.experimental.pallas import tpu_sc as plsc

B, N, D = 16384, 200, 128


def _sc_gather(emb_flat, idx):
    info = plsc.get_sparse_core_info()
    NC, NS, L = info.num_cores, info.num_subcores, info.num_lanes
    NW = NC * NS
    b_per_w = B // NW
    CH = 128
    n_ch = b_per_w // CH
    mesh = plsc.VectorSubcoreMesh(core_axis_name="c", subcore_axis_name="s")

    @functools.partial(
        pl.kernel,
        mesh=mesh,
        out_type=jax.ShapeDtypeStruct((B, D), jnp.float32),
        scratch_types=[
            pltpu.VMEM((b_per_w,), jnp.int32),
            pltpu.VMEM((n_ch, CH), jnp.int32),
            pltpu.VMEM((b_per_w, D), jnp.float32),
            pltpu.SemaphoreType.DMA,
            pltpu.SemaphoreType.DMA,
        ],
    )
    def k(emb_hbm, idx_hbm, out_hbm, idx_raw, idx_v, rows_v, gsem, wsem):
        wid = lax.axis_index("s") * NC + lax.axis_index("c")
        base = wid * b_per_w
        pltpu.sync_copy(idx_hbm.at[pl.ds(base, b_per_w)], idx_raw)
        lane_off = lax.iota(jnp.int32, L) * N
        per_ch = CH // L
        gathers = []
        for c in range(n_ch):
            for jj in range(per_ch):
                j = c * per_ch + jj
                row0 = (base + j * L) * N
                v = idx_raw[pl.ds(j * L, L)] + (lane_off + row0)
                idx_v[c, pl.ds(jj * L, L)] = v
            cp = pltpu.make_async_copy(
                emb_hbm.at[idx_v.at[c]], rows_v.at[pl.ds(c * CH, CH)], gsem
            )
            cp.start()
            gathers.append(cp)
        writes = []
        for c in range(n_ch):
            gathers[c].wait()
            wr = pltpu.make_async_copy(
                rows_v.at[pl.ds(c * CH, CH)], out_hbm.at[pl.ds(base + c * CH, CH)], wsem
            )
            wr.start()
            writes.append(wr)
        for wr in writes:
            wr.wait()

    return k(emb_flat, idx)


def _tc_project(g, ft, w0, wfx, bias):
    BLK = 2048
    grid = (B // BLK,)

    def body(g_ref, ft_ref, w0_ref, wfx_ref, b_ref, o_ref):
        acc = jnp.dot(g_ref[...], w0_ref[...], preferred_element_type=jnp.float32)
        acc += lax.dot_general(
            ft_ref[...], wfx_ref[...], (((0,), (0,)), ((), ())),
            preferred_element_type=jnp.float32,
        )
        o_ref[...] = acc + b_ref[...]

    row = lambda i: (i, 0)
    fcol = lambda i: (0, i)
    fixed = lambda i: (0, 0)
    return pl.pallas_call(
        body,
        grid=grid,
        in_specs=[
            pl.BlockSpec((BLK, D), row),
            pl.BlockSpec((4, BLK), fcol),
            pl.BlockSpec((D, D), fixed),
            pl.BlockSpec((4, D), fixed),
            pl.BlockSpec((1, D), fixed),
        ],
        out_specs=pl.BlockSpec((BLK, D), row),
        out_shape=jax.ShapeDtypeStruct((B, D), jnp.float32),
    )(g, ft, w0, wfx, bias)


def kernel(embeddings, current_node, vehicle_capacity, used_capacity, current_time, i, W, b):
    emb_flat = embeddings.reshape(B * N, D)
    idx = current_node.astype(jnp.int32)
    g = _sc_gather(emb_flat, idx)
    w0 = W[:D]
    ft = jnp.concatenate(
        [vehicle_capacity.T, used_capacity.T, current_time.T, i.T], axis=0
    )
    wfx = jnp.concatenate([W[D:D + 1], -W[D:D + 1], W[D + 1:D + 2], W[D + 2:D + 3]], axis=0)
    bias = b.reshape(1, D)
    return _tc_project(g, ft, w0, wfx, bias)
